# baseline (speedup 1.0000x reference)
import jax
import jax.numpy as jnp
import numpy as np
from jax import lax
from jax.experimental import pallas as pl
from jax.experimental.pallas import tpu as pltpu
from jax.experimental.pallas import tpu_sc as plsc

B = 16384
HW = 49
CIN = 3
COUT = 20
ROW = HW * CIN
OUT_ROW = COUT * HW

NC, NS, L = 2, 16, 16
NW = NC * NS
B_PER_W = B // NW
GB = 4
VPG = GB * OUT_ROW // L
SUBS = 8
SG = GB * SUBS
SVPG = VPG * SUBS
NGROUPS = B_PER_W // SG


def _tables():
    f = np.arange(GB * OUT_ROW)
    b = f // OUT_ROW
    r = f % OUT_ROW
    c = r // HW
    p = r % HW
    ch = np.where(c < 11, 0, np.where(c < 17, 1, 2))
    loc = np.where(c < 11, c, np.where(c < 17, c - 11, c - 17))
    idx = b * ROW + p * CIN + ch
    return (idx.reshape(VPG, L).astype(np.int32),
            loc.reshape(VPG, L).astype(np.int32))


_IDX_TAB, _CLS_TAB = _tables()


def _sc_body(x_hbm, idx_hbm, cls_hbm, out_hbm, x_v, idx_v, cls_v, out_v):
    wid = lax.axis_index("s") * NC + lax.axis_index("c")
    pltpu.sync_copy(idx_hbm, idx_v)
    pltpu.sync_copy(cls_hbm, cls_v)
    pltpu.sync_copy(x_hbm.at[pl.ds(wid * (B_PER_W * ROW), B_PER_W * ROW)], x_v)
    out_row0 = wid * (B_PER_W * OUT_ROW // L)

    def group(g, carry):
        def sub(s, c1):
            off = g * (SG * ROW) + s * (GB * ROW)
            row0 = s * VPG

            def vec(v, c2):
                iv = idx_v[v] + off
                xv = plsc.load_gather(x_v, [iv])
                cv = cls_v[v]
                out_v[row0 + v] = jnp.where(
                    xv == cv, jnp.float32(1.0), jnp.float32(0.0))
                return c2

            lax.fori_loop(0, VPG, vec, None, unroll=2)
            return c1

        lax.fori_loop(0, SUBS, sub, None)
        pltpu.sync_copy(out_v, out_hbm.at[pl.ds(out_row0 + g * SVPG, SVPG)])
        return carry

    lax.fori_loop(0, NGROUPS, group, None)


_SC_CALL = None


def _get_sc_call():
    global _SC_CALL
    if _SC_CALL is None:
        mesh = plsc.VectorSubcoreMesh(
            core_axis_name="c", subcore_axis_name="s",
            num_cores=NC, num_subcores=NS)
        _SC_CALL = pl.kernel(
            _sc_body,
            out_type=jax.ShapeDtypeStruct((B * OUT_ROW // L, L), jnp.float32),
            mesh=mesh,
            scratch_types=[
                pltpu.VMEM((B_PER_W * ROW,), jnp.int32),
                pltpu.VMEM((VPG, L), jnp.int32),
                pltpu.VMEM((VPG, L), jnp.int32),
                pltpu.VMEM((SVPG, L), jnp.float32),
            ],
            compiler_params=pltpu.CompilerParams(
                needs_layout_passes=False, use_tc_tiling_on_sc=False),
        )
    return _SC_CALL


def kernel(x):
    xf = x.reshape(B * ROW)
    out = _get_sc_call()(xf, jnp.asarray(_IDX_TAB), jnp.asarray(_CLS_TAB))
    return out.reshape(B, COUT, 7, 7)
